# baseline (speedup 1.0000x reference)
import functools

import jax
import jax.numpy as jnp
from jax import lax
from jax.experimental import pallas as pl
from jax.experimental.pallas import tpu as pltpu, tpu_sc as plsc

N = 10000
E = 320000
D = 128
DH = D // 2

NC = 2
NS = 16
CHUNK = 128
NCH = 160
E_PAD = NS * NCH * CHUNK
SINK = N
R = 10112
RPT = R // NS
BLK = 400
GRID = N // BLK
CW = 8
NB = 4


def _sc_scatter(with_counts):
    out_type = [jax.ShapeDtypeStruct((NC, R, DH), jnp.float32)]
    if with_counts:
        out_type.append(jax.ShapeDtypeStruct((NC, R, CW), jnp.float32))
    scratch = [
        pltpu.VMEM((NCH, CHUNK), jnp.int32),
        pltpu.VMEM((NCH, CHUNK), jnp.int32),
        pltpu.VMEM((NB, CHUNK, DH), jnp.float32),
        pltpu.VMEM((CHUNK, CW), jnp.float32),
        pltpu.VMEM_SHARED((R, DH), jnp.float32),
        pltpu.VMEM_SHARED((R, CW), jnp.float32),
    ] + [pltpu.SemaphoreType.DMA] * (2 * NB)

    mesh = plsc.VectorSubcoreMesh(core_axis_name="c", subcore_axis_name="s")

    @functools.partial(
        pl.kernel, out_type=tuple(out_type), mesh=mesh,
        scratch_types=scratch,
        compiler_params=pltpu.CompilerParams(use_tc_tiling_on_sc=False))
    def k(z_hbm, src_hbm, dst_hbm, zeros_hbm, zeros_c_hbm, ones_hbm,
          part_hbm, *rest):
        if with_counts:
            cnt_hbm = rest[0]
            rest = rest[1:]
        src_v, dst_v, rows_v, ones_v, acc, cacc = rest[:6]
        sems_g = rest[6:6 + NB]
        sems_s = rest[6 + NB:]
        c = lax.axis_index("c")
        s = lax.axis_index("s")

        pltpu.sync_copy(zeros_hbm, acc.at[pl.ds(s * RPT, RPT)])
        if with_counts:
            pltpu.sync_copy(zeros_c_hbm, cacc.at[pl.ds(s * RPT, RPT)])
            pltpu.sync_copy(ones_hbm, ones_v)
        pltpu.sync_copy(src_hbm.at[s], src_v)
        pltpu.sync_copy(dst_hbm.at[s], dst_v)
        plsc.subcore_barrier()

        zc = z_hbm.at[c]

        for b in range(NB):
            pltpu.async_copy(zc.at[src_v.at[b]], rows_v.at[b], sems_g[b])

        @pl.loop(0, NCH, step=NB)
        def chunk_loop(j):
            for b in range(NB):
                jj = j + b
                pltpu.make_async_copy(
                    zc.at[src_v.at[jj]], rows_v.at[b], sems_g[b]).wait()
                pltpu.async_copy(rows_v.at[b], acc.at[dst_v.at[jj]],
                                 sems_s[b], add=True)
                if with_counts:
                    @pl.when(c == b % NC)
                    def _():
                        pltpu.sync_copy(ones_v, cacc.at[dst_v.at[jj]],
                                        add=True)
                nxt = jj + NB

                @pl.when(nxt < NCH)
                def _():
                    pltpu.make_async_copy(
                        rows_v.at[b], acc.at[dst_v.at[jj]], sems_s[b]).wait()
                    pltpu.async_copy(zc.at[src_v.at[nxt]], rows_v.at[b],
                                     sems_g[b])

        for b in range(NB):
            jj = NCH - NB + b
            pltpu.make_async_copy(
                rows_v.at[b], acc.at[dst_v.at[jj]], sems_s[b]).wait()

        plsc.subcore_barrier()
        rows = pl.ds(s * RPT, RPT)
        pltpu.sync_copy(acc.at[rows], part_hbm.at[c, rows])
        if with_counts:
            pltpu.sync_copy(cacc.at[rows], cnt_hbm.at[c, rows])

    return k


_sc_scatter_l1 = _sc_scatter(True)
_sc_scatter_l2 = _sc_scatter(False)


def _dotT(a, w):
    return lax.dot_general(a, w, (((1,), (1,)), ((), ())),
                           preferred_element_type=jnp.float32)


def _tc_layer1(x, Wl, Wr, b):
    def body(x_ref, wl_ref, wr_ref, b_ref, z_ref, y_ref):
        x = x_ref[...]
        zl = _dotT(x, wl_ref[...])
        z_ref[0] = zl[:, :DH]
        z_ref[1] = zl[:, DH:]
        y_ref[...] = _dotT(x, wr_ref[...]) + b_ref[...]

    z, y = pl.pallas_call(
        body,
        grid=(GRID,),
        in_specs=[
            pl.BlockSpec((BLK, D), lambda i: (i, 0)),
            pl.BlockSpec((D, D), lambda i: (0, 0)),
            pl.BlockSpec((D, D), lambda i: (0, 0)),
            pl.BlockSpec((1, D), lambda i: (0, 0)),
        ],
        out_specs=[
            pl.BlockSpec((NC, BLK, DH), lambda i: (0, i, 0)),
            pl.BlockSpec((BLK, D), lambda i: (i, 0)),
        ],
        out_shape=[
            jax.ShapeDtypeStruct((NC, N, DH), jnp.float32),
            jax.ShapeDtypeStruct((N, D), jnp.float32),
        ],
    )(x, Wl, Wr, b.reshape(1, D))
    return z, y


def _tc_mid(p, cnt, y1, Wl, Wr, b):
    def body(p_ref, c_ref, y1_ref, wl_ref, wr_ref, b_ref, z_ref, y_ref):
        agg = jnp.concatenate([p_ref[0], p_ref[1]], axis=1)
        cn = c_ref[0, :, :1] + c_ref[1, :, :1]
        inv = 1.0 / jnp.maximum(cn, 1.0)
        h = jnp.maximum(agg * inv + y1_ref[...], 0.0)
        zl = _dotT(h, wl_ref[...])
        z_ref[0] = zl[:, :DH]
        z_ref[1] = zl[:, DH:]
        y_ref[...] = _dotT(h, wr_ref[...]) + b_ref[...]

    z2, y2 = pl.pallas_call(
        body,
        grid=(GRID,),
        in_specs=[
            pl.BlockSpec((NC, BLK, DH), lambda i: (0, i, 0)),
            pl.BlockSpec((NC, BLK, CW), lambda i: (0, i, 0)),
            pl.BlockSpec((BLK, D), lambda i: (i, 0)),
            pl.BlockSpec((D, D), lambda i: (0, 0)),
            pl.BlockSpec((D, D), lambda i: (0, 0)),
            pl.BlockSpec((1, D), lambda i: (0, 0)),
        ],
        out_specs=[
            pl.BlockSpec((NC, BLK, DH), lambda i: (0, i, 0)),
            pl.BlockSpec((BLK, D), lambda i: (i, 0)),
        ],
        out_shape=[
            jax.ShapeDtypeStruct((NC, N, DH), jnp.float32),
            jax.ShapeDtypeStruct((N, D), jnp.float32),
        ],
    )(p, cnt, y1, Wl, Wr, b.reshape(1, D))
    return z2, y2


def _tc_out(q, cnt, y2):
    def body(q_ref, c_ref, y2_ref, o_ref):
        agg = jnp.concatenate([q_ref[0], q_ref[1]], axis=1)
        cn = c_ref[0, :, :1] + c_ref[1, :, :1]
        inv = 1.0 / jnp.maximum(cn, 1.0)
        o_ref[...] = agg * inv + y2_ref[...]

    return pl.pallas_call(
        body,
        grid=(GRID,),
        in_specs=[
            pl.BlockSpec((NC, BLK, DH), lambda i: (0, i, 0)),
            pl.BlockSpec((NC, BLK, CW), lambda i: (0, i, 0)),
            pl.BlockSpec((BLK, D), lambda i: (i, 0)),
        ],
        out_specs=pl.BlockSpec((BLK, D), lambda i: (i, 0)),
        out_shape=jax.ShapeDtypeStruct((N, D), jnp.float32),
    )(q, cnt, y2)


def kernel(x, edge_index, W1l, b1l, W1r, W2l, b2l, W2r):
    src = edge_index[0]
    dst = edge_index[1]
    pad = E_PAD - E
    src_p = jnp.concatenate([src, jnp.zeros((pad,), src.dtype)]
                            ).reshape(NS, NCH, CHUNK).astype(jnp.int32)
    dst_p = jnp.concatenate([dst, jnp.full((pad,), SINK, dst.dtype)]
                            ).reshape(NS, NCH, CHUNK).astype(jnp.int32)
    zeros = jnp.zeros((RPT, DH), jnp.float32)
    zeros_c = jnp.zeros((RPT, CW), jnp.float32)
    ones = jnp.ones((CHUNK, CW), jnp.float32)

    z1, y1 = _tc_layer1(x, W1l, W1r, b1l)
    p, cnt = _sc_scatter_l1(z1, src_p, dst_p, zeros, zeros_c, ones)
    z2, y2 = _tc_mid(p, cnt, y1, W2l, W2r, b2l)
    (q,) = _sc_scatter_l2(z2, src_p, dst_p, zeros, zeros_c, ones)
    return _tc_out(q, cnt, y2)


# speedup vs baseline: 2.2952x; 2.2952x over previous
import functools

import jax
import jax.numpy as jnp
from jax import lax
from jax.experimental import pallas as pl
from jax.experimental.pallas import tpu as pltpu, tpu_sc as plsc

N = 10000
E = 320000
D = 128
DH = D // 2

NC = 2
NS = 16
CHUNK = 128
NCH = 160
E_PAD = NS * NCH * CHUNK
SINK = N
R = 10112
RPT = R // NS
BLK = 400
GRID = N // BLK
CW = 8
NB = 4


def _sc_scatter(with_counts):
    out_type = [jax.ShapeDtypeStruct((NC, R, DH), jnp.float32)]
    if with_counts:
        out_type.append(jax.ShapeDtypeStruct((NC, R, CW), jnp.float32))
    scratch = [
        pltpu.VMEM((NCH, CHUNK), jnp.int32),
        pltpu.VMEM((NCH, CHUNK), jnp.int32),
        pltpu.VMEM((NB, CHUNK, DH), jnp.float32),
        pltpu.VMEM((CHUNK, CW), jnp.float32),
        pltpu.VMEM_SHARED((R, DH), jnp.float32),
        pltpu.VMEM_SHARED((R, CW), jnp.float32),
    ] + [pltpu.SemaphoreType.DMA] * NB

    mesh = plsc.VectorSubcoreMesh(core_axis_name="c", subcore_axis_name="s")

    @functools.partial(
        pl.kernel, out_type=tuple(out_type), mesh=mesh,
        scratch_types=scratch,
        compiler_params=pltpu.CompilerParams(use_tc_tiling_on_sc=False))
    def k(z_hbm, src_hbm, dst_hbm, zeros_hbm, zeros_c_hbm, ones_hbm,
          part_hbm, *rest):
        if with_counts:
            cnt_hbm = rest[0]
            rest = rest[1:]
        src_v, dst_v, rows_v, ones_v, acc, cacc = rest[:6]
        sems_g = rest[6:]
        c = lax.axis_index("c")
        s = lax.axis_index("s")

        pltpu.sync_copy(zeros_hbm, acc.at[pl.ds(s * RPT, RPT)])
        if with_counts:
            pltpu.sync_copy(zeros_c_hbm, cacc.at[pl.ds(s * RPT, RPT)])
            pltpu.sync_copy(ones_hbm, ones_v)
        pltpu.sync_copy(src_hbm.at[s], src_v)
        pltpu.sync_copy(dst_hbm.at[s], dst_v)
        plsc.subcore_barrier()

        zc = z_hbm.at[c]

        for b in range(NB):
            pltpu.async_copy(zc.at[src_v.at[b]], rows_v.at[b], sems_g[b])

        @pl.loop(0, NCH, step=NB)
        def chunk_loop(j):
            for b in range(NB):
                jj = j + b
                pltpu.make_async_copy(
                    zc.at[src_v.at[jj]], rows_v.at[b], sems_g[b]).wait()
                pltpu.sync_copy(rows_v.at[b], acc.at[dst_v.at[jj]], add=True)
                if with_counts:
                    @pl.when(c == b % NC)
                    def _():
                        pltpu.sync_copy(ones_v, cacc.at[dst_v.at[jj]],
                                        add=True)
                nxt = jj + NB

                @pl.when(nxt < NCH)
                def _():
                    pltpu.async_copy(zc.at[src_v.at[nxt]], rows_v.at[b],
                                     sems_g[b])

        plsc.subcore_barrier()
        rows = pl.ds(s * RPT, RPT)
        pltpu.sync_copy(acc.at[rows], part_hbm.at[c, rows])
        if with_counts:
            pltpu.sync_copy(cacc.at[rows], cnt_hbm.at[c, rows])

    return k


_sc_scatter_l1 = _sc_scatter(True)
_sc_scatter_l2 = _sc_scatter(False)


def _dotT(a, w):
    return lax.dot_general(a, w, (((1,), (1,)), ((), ())),
                           preferred_element_type=jnp.float32)


def _tc_layer1(x, Wl, Wr, b):
    def body(x_ref, wl_ref, wr_ref, b_ref, z_ref, y_ref):
        x = x_ref[...]
        zl = _dotT(x, wl_ref[...])
        z_ref[0] = zl[:, :DH]
        z_ref[1] = zl[:, DH:]
        y_ref[...] = _dotT(x, wr_ref[...]) + b_ref[...]

    z, y = pl.pallas_call(
        body,
        grid=(GRID,),
        in_specs=[
            pl.BlockSpec((BLK, D), lambda i: (i, 0)),
            pl.BlockSpec((D, D), lambda i: (0, 0)),
            pl.BlockSpec((D, D), lambda i: (0, 0)),
            pl.BlockSpec((1, D), lambda i: (0, 0)),
        ],
        out_specs=[
            pl.BlockSpec((NC, BLK, DH), lambda i: (0, i, 0)),
            pl.BlockSpec((BLK, D), lambda i: (i, 0)),
        ],
        out_shape=[
            jax.ShapeDtypeStruct((NC, N, DH), jnp.float32),
            jax.ShapeDtypeStruct((N, D), jnp.float32),
        ],
    )(x, Wl, Wr, b.reshape(1, D))
    return z, y


def _tc_mid(p, cnt, y1, Wl, Wr, b):
    def body(p_ref, c_ref, y1_ref, wl_ref, wr_ref, b_ref, z_ref, y_ref):
        agg = jnp.concatenate([p_ref[0], p_ref[1]], axis=1)
        cn = c_ref[0, :, :1] + c_ref[1, :, :1]
        inv = 1.0 / jnp.maximum(cn, 1.0)
        h = jnp.maximum(agg * inv + y1_ref[...], 0.0)
        zl = _dotT(h, wl_ref[...])
        z_ref[0] = zl[:, :DH]
        z_ref[1] = zl[:, DH:]
        y_ref[...] = _dotT(h, wr_ref[...]) + b_ref[...]

    z2, y2 = pl.pallas_call(
        body,
        grid=(GRID,),
        in_specs=[
            pl.BlockSpec((NC, BLK, DH), lambda i: (0, i, 0)),
            pl.BlockSpec((NC, BLK, CW), lambda i: (0, i, 0)),
            pl.BlockSpec((BLK, D), lambda i: (i, 0)),
            pl.BlockSpec((D, D), lambda i: (0, 0)),
            pl.BlockSpec((D, D), lambda i: (0, 0)),
            pl.BlockSpec((1, D), lambda i: (0, 0)),
        ],
        out_specs=[
            pl.BlockSpec((NC, BLK, DH), lambda i: (0, i, 0)),
            pl.BlockSpec((BLK, D), lambda i: (i, 0)),
        ],
        out_shape=[
            jax.ShapeDtypeStruct((NC, N, DH), jnp.float32),
            jax.ShapeDtypeStruct((N, D), jnp.float32),
        ],
    )(p, cnt, y1, Wl, Wr, b.reshape(1, D))
    return z2, y2


def _tc_out(q, cnt, y2):
    def body(q_ref, c_ref, y2_ref, o_ref):
        agg = jnp.concatenate([q_ref[0], q_ref[1]], axis=1)
        cn = c_ref[0, :, :1] + c_ref[1, :, :1]
        inv = 1.0 / jnp.maximum(cn, 1.0)
        o_ref[...] = agg * inv + y2_ref[...]

    return pl.pallas_call(
        body,
        grid=(GRID,),
        in_specs=[
            pl.BlockSpec((NC, BLK, DH), lambda i: (0, i, 0)),
            pl.BlockSpec((NC, BLK, CW), lambda i: (0, i, 0)),
            pl.BlockSpec((BLK, D), lambda i: (i, 0)),
        ],
        out_specs=pl.BlockSpec((BLK, D), lambda i: (i, 0)),
        out_shape=jax.ShapeDtypeStruct((N, D), jnp.float32),
    )(q, cnt, y2)


def kernel(x, edge_index, W1l, b1l, W1r, W2l, b2l, W2r):
    src = edge_index[0]
    dst = edge_index[1]
    pad = E_PAD - E
    pad_src = (jnp.arange(pad, dtype=src.dtype) * 13) % N
    pad_dst = SINK + (jnp.arange(pad, dtype=dst.dtype) % (R - SINK))
    src_p = jnp.concatenate([src, pad_src]
                            ).reshape(NS, NCH, CHUNK).astype(jnp.int32)
    dst_p = jnp.concatenate([dst, pad_dst]
                            ).reshape(NS, NCH, CHUNK).astype(jnp.int32)
    zeros = jnp.zeros((RPT, DH), jnp.float32)
    zeros_c = jnp.zeros((RPT, CW), jnp.float32)
    ones = jnp.ones((CHUNK, CW), jnp.float32)

    z1, y1 = _tc_layer1(x, W1l, W1r, b1l)
    p, cnt = _sc_scatter_l1(z1, src_p, dst_p, zeros, zeros_c, ones)
    z2, y2 = _tc_mid(p, cnt, y1, W2l, W2r, b2l)
    (q,) = _sc_scatter_l2(z2, src_p, dst_p, zeros, zeros_c, ones)
    return _tc_out(q, cnt, y2)


# speedup vs baseline: 2.5255x; 1.1003x over previous
import functools

import jax
import jax.numpy as jnp
from jax import lax
from jax.experimental import pallas as pl
from jax.experimental.pallas import tpu as pltpu, tpu_sc as plsc

N = 10000
E = 320000
D = 128
DH = D // 2

NC = 2
NS = 16
CHUNK = 128
NCH = 160
E_PAD = NS * NCH * CHUNK
SINK = N
R = 10112
RPT = R // NS
BLK = 2000
GRID = N // BLK
CW = 8
NB = 4


def _sc_scatter(with_counts):
    out_type = [jax.ShapeDtypeStruct((NC, R, DH), jnp.float32)]
    if with_counts:
        out_type.append(jax.ShapeDtypeStruct((NC, R, CW), jnp.float32))
    scratch = [
        pltpu.VMEM((NCH, CHUNK), jnp.int32),
        pltpu.VMEM((NCH, CHUNK), jnp.int32),
        pltpu.VMEM((NB, CHUNK, DH), jnp.float32),
        pltpu.VMEM((CHUNK, CW), jnp.float32),
        pltpu.VMEM_SHARED((R, DH), jnp.float32),
        pltpu.VMEM_SHARED((R, CW), jnp.float32),
    ] + [pltpu.SemaphoreType.DMA] * NB

    mesh = plsc.VectorSubcoreMesh(core_axis_name="c", subcore_axis_name="s")

    @functools.partial(
        pl.kernel, out_type=tuple(out_type), mesh=mesh,
        scratch_types=scratch,
        compiler_params=pltpu.CompilerParams(use_tc_tiling_on_sc=False))
    def k(z_hbm, src_hbm, dst_hbm, zeros_hbm, zeros_c_hbm, ones_hbm,
          part_hbm, *rest):
        if with_counts:
            cnt_hbm = rest[0]
            rest = rest[1:]
        src_v, dst_v, rows_v, ones_v, acc, cacc = rest[:6]
        sems_g = rest[6:]
        c = lax.axis_index("c")
        s = lax.axis_index("s")

        pltpu.sync_copy(zeros_hbm, acc.at[pl.ds(s * RPT, RPT)])
        if with_counts:
            pltpu.sync_copy(zeros_c_hbm, cacc.at[pl.ds(s * RPT, RPT)])
            pltpu.sync_copy(ones_hbm, ones_v)
        pltpu.sync_copy(src_hbm.at[s], src_v)
        pltpu.sync_copy(dst_hbm.at[s], dst_v)
        plsc.subcore_barrier()

        zc = z_hbm.at[c]

        for b in range(NB):
            pltpu.async_copy(zc.at[src_v.at[b]], rows_v.at[b], sems_g[b])

        @pl.loop(0, NCH, step=NB)
        def chunk_loop(j):
            for b in range(NB):
                jj = j + b
                pltpu.make_async_copy(
                    zc.at[src_v.at[jj]], rows_v.at[b], sems_g[b]).wait()
                pltpu.sync_copy(rows_v.at[b], acc.at[dst_v.at[jj]], add=True)
                if with_counts:
                    @pl.when(c == b % NC)
                    def _():
                        pltpu.sync_copy(ones_v, cacc.at[dst_v.at[jj]],
                                        add=True)
                nxt = jj + NB

                @pl.when(nxt < NCH)
                def _():
                    pltpu.async_copy(zc.at[src_v.at[nxt]], rows_v.at[b],
                                     sems_g[b])

        plsc.subcore_barrier()
        rows = pl.ds(s * RPT, RPT)
        pltpu.sync_copy(acc.at[rows], part_hbm.at[c, rows])
        if with_counts:
            pltpu.sync_copy(cacc.at[rows], cnt_hbm.at[c, rows])

    return k


_sc_scatter_l1 = _sc_scatter(True)
_sc_scatter_l2 = _sc_scatter(False)


def _dotT(a, w):
    return lax.dot_general(a, w, (((1,), (1,)), ((), ())),
                           preferred_element_type=jnp.float32)


def _tc_layer1(x, Wl, Wr, b):
    def body(x_ref, wl_ref, wr_ref, b_ref, z_ref, y_ref):
        x = x_ref[...]
        zl = _dotT(x, wl_ref[...])
        z_ref[0] = zl[:, :DH]
        z_ref[1] = zl[:, DH:]
        y_ref[...] = _dotT(x, wr_ref[...]) + b_ref[...]

    z, y = pl.pallas_call(
        body,
        grid=(GRID,),
        in_specs=[
            pl.BlockSpec((BLK, D), lambda i: (i, 0)),
            pl.BlockSpec((D, D), lambda i: (0, 0)),
            pl.BlockSpec((D, D), lambda i: (0, 0)),
            pl.BlockSpec((1, D), lambda i: (0, 0)),
        ],
        out_specs=[
            pl.BlockSpec((NC, BLK, DH), lambda i: (0, i, 0)),
            pl.BlockSpec((BLK, D), lambda i: (i, 0)),
        ],
        out_shape=[
            jax.ShapeDtypeStruct((NC, N, DH), jnp.float32),
            jax.ShapeDtypeStruct((N, D), jnp.float32),
        ],
    )(x, Wl, Wr, b.reshape(1, D))
    return z, y


def _tc_mid(p, cnt, y1, Wl, Wr, b):
    def body(p_ref, c_ref, y1_ref, wl_ref, wr_ref, b_ref, z_ref, y_ref):
        agg = jnp.concatenate([p_ref[0], p_ref[1]], axis=1)
        cn = c_ref[0, :, :1] + c_ref[1, :, :1]
        inv = 1.0 / jnp.maximum(cn, 1.0)
        h = jnp.maximum(agg * inv + y1_ref[...], 0.0)
        zl = _dotT(h, wl_ref[...])
        z_ref[0] = zl[:, :DH]
        z_ref[1] = zl[:, DH:]
        y_ref[...] = _dotT(h, wr_ref[...]) + b_ref[...]

    z2, y2 = pl.pallas_call(
        body,
        grid=(GRID,),
        in_specs=[
            pl.BlockSpec((NC, BLK, DH), lambda i: (0, i, 0)),
            pl.BlockSpec((NC, BLK, CW), lambda i: (0, i, 0)),
            pl.BlockSpec((BLK, D), lambda i: (i, 0)),
            pl.BlockSpec((D, D), lambda i: (0, 0)),
            pl.BlockSpec((D, D), lambda i: (0, 0)),
            pl.BlockSpec((1, D), lambda i: (0, 0)),
        ],
        out_specs=[
            pl.BlockSpec((NC, BLK, DH), lambda i: (0, i, 0)),
            pl.BlockSpec((BLK, D), lambda i: (i, 0)),
        ],
        out_shape=[
            jax.ShapeDtypeStruct((NC, N, DH), jnp.float32),
            jax.ShapeDtypeStruct((N, D), jnp.float32),
        ],
    )(p, cnt, y1, Wl, Wr, b.reshape(1, D))
    return z2, y2


def _tc_out(q, cnt, y2):
    def body(q_ref, c_ref, y2_ref, o_ref):
        agg = jnp.concatenate([q_ref[0], q_ref[1]], axis=1)
        cn = c_ref[0, :, :1] + c_ref[1, :, :1]
        inv = 1.0 / jnp.maximum(cn, 1.0)
        o_ref[...] = agg * inv + y2_ref[...]

    return pl.pallas_call(
        body,
        grid=(GRID,),
        in_specs=[
            pl.BlockSpec((NC, BLK, DH), lambda i: (0, i, 0)),
            pl.BlockSpec((NC, BLK, CW), lambda i: (0, i, 0)),
            pl.BlockSpec((BLK, D), lambda i: (i, 0)),
        ],
        out_specs=pl.BlockSpec((BLK, D), lambda i: (i, 0)),
        out_shape=jax.ShapeDtypeStruct((N, D), jnp.float32),
    )(q, cnt, y2)


def kernel(x, edge_index, W1l, b1l, W1r, W2l, b2l, W2r):
    src = edge_index[0]
    dst = edge_index[1]
    pad = E_PAD - E
    pad_src = (jnp.arange(pad, dtype=src.dtype) * 13) % N
    pad_dst = SINK + (jnp.arange(pad, dtype=dst.dtype) % (R - SINK))
    src_p = jnp.concatenate([src, pad_src]
                            ).reshape(NS, NCH, CHUNK).astype(jnp.int32)
    dst_p = jnp.concatenate([dst, pad_dst]
                            ).reshape(NS, NCH, CHUNK).astype(jnp.int32)
    zeros = jnp.zeros((RPT, DH), jnp.float32)
    zeros_c = jnp.zeros((RPT, CW), jnp.float32)
    ones = jnp.ones((CHUNK, CW), jnp.float32)

    z1, y1 = _tc_layer1(x, W1l, W1r, b1l)
    p, cnt = _sc_scatter_l1(z1, src_p, dst_p, zeros, zeros_c, ones)
    z2, y2 = _tc_mid(p, cnt, y1, W2l, W2r, b2l)
    (q,) = _sc_scatter_l2(z2, src_p, dst_p, zeros, zeros_c, ones)
    return _tc_out(q, cnt, y2)
